# Initial kernel scaffold; baseline (speedup 1.0000x reference)
import functools

import jax
import jax.numpy as jnp
from jax import lax
from jax.experimental import pallas as pl
from jax.experimental.pallas import tpu as pltpu
from jax.experimental.pallas import tpu_sc as plsc

BATCH = 16384
FIELDS = 26
OUT_DIM = 32
TOTAL = BATCH * FIELDS

_NC = 2
_NS = 16
_NW = _NC * _NS
_B_PER_W = TOTAL // _NW
_CHUNK = 1024
_N_CHUNKS = _B_PER_W // _CHUNK

_mesh = plsc.VectorSubcoreMesh(core_axis_name="c", subcore_axis_name="s")


@functools.partial(
    pl.kernel,
    mesh=_mesh,
    out_type=jax.ShapeDtypeStruct((TOTAL, OUT_DIM), jnp.float32),
    scratch_types=[
        pltpu.VMEM((_CHUNK,), jnp.int32),
        pltpu.VMEM((_CHUNK, OUT_DIM), jnp.float32),
        pltpu.SemaphoreType.DMA,
    ],
)
def _gather_kernel(idx_hbm, table_hbm, out_hbm, idx_v, rows_v, sem):
    wid = lax.axis_index("s") * _NC + lax.axis_index("c")
    base = wid * _B_PER_W
    for g in range(_N_CHUNKS):
        off = base + g * _CHUNK
        pltpu.sync_copy(idx_hbm.at[pl.ds(off, _CHUNK)], idx_v)
        pltpu.async_copy(table_hbm.at[idx_v], rows_v, sem).wait()
        pltpu.sync_copy(rows_v, out_hbm.at[pl.ds(off, _CHUNK)])


def kernel(input, table):
    flat = input.reshape(TOTAL)
    out = _gather_kernel(flat, table)
    return out.reshape(BATCH, FIELDS, OUT_DIM)


# speedup vs baseline: 1.3641x; 1.3641x over previous
import functools

import jax
import jax.numpy as jnp
from jax import lax
from jax.experimental import pallas as pl
from jax.experimental.pallas import tpu as pltpu
from jax.experimental.pallas import tpu_sc as plsc

BATCH = 16384
FIELDS = 26
OUT_DIM = 32
ROWS4 = 250000

_NC = 2
_NS = 16
_NW = _NC * _NS
_BW = BATCH // _NW
_SUB = 128
_NSUB = _BW // _SUB

_mesh = plsc.VectorSubcoreMesh(core_axis_name="c", subcore_axis_name="s")


@functools.partial(
    pl.kernel,
    mesh=_mesh,
    out_type=jax.ShapeDtypeStruct((FIELDS, OUT_DIM, BATCH), jnp.float32),
    scratch_types=[
        pltpu.VMEM((2, _BW), jnp.int32),
        pltpu.VMEM((2, _NSUB, _SUB), jnp.int32),
        pltpu.VMEM((2, _SUB, 128), jnp.float32),
        pltpu.VMEM((2, OUT_DIM, _BW), jnp.float32),
        pltpu.SemaphoreType.DMA((2,)),
        pltpu.SemaphoreType.DMA((2,)),
    ],
    compiler_params=pltpu.CompilerParams(use_tc_tiling_on_sc=True, needs_layout_passes=False),
)
def _gather_kernel(idx_hbm, table_hbm, out_hbm, idx_v, row_v, rows_v, osb,
                   gsem, osem):
    wid = lax.axis_index("s") * _NC + lax.axis_index("c")
    b0 = wid * _BW

    iota = lax.iota(jnp.int32, 16)

    def load_field_idx(f, ob):
        pltpu.sync_copy(idx_hbm.at[f, pl.ds(b0, _BW)], idx_v.at[ob])
        for sub in range(_NSUB):
            for g in range(_SUB // 16):
                v = idx_v[ob, pl.ds(sub * _SUB + g * 16, 16)]
                row_v[ob, sub, pl.ds(g * 16, 16)] = (
                    lax.shift_right_logical(v, 2))

    def gat(sub, gb, ob):
        return pltpu.make_async_copy(
            table_hbm.at[row_v.at[ob, sub]],
            rows_v.at[gb], gsem.at[gb])

    def outw(f, ob):
        return pltpu.make_async_copy(
            osb.at[ob], out_hbm.at[f, :, pl.ds(b0, _BW)], osem.at[ob])

    def extract(sub, gb, ob):
        base_b = sub * _SUB
        for g in range(_SUB // 16):
            idx16 = idx_v[ob, pl.ds(base_b + g * 16, 16)]
            lane0 = lax.shift_left(lax.bitwise_and(idx16, jnp.int32(3)),
                                   jnp.int32(5))
            rows16 = iota + jnp.int32(g * 16)
            for d in range(OUT_DIM):
                vals = plsc.load_gather(
                    rows_v.at[gb], [rows16, lane0 + jnp.int32(d)])
                osb[ob, d, pl.ds(base_b + g * 16, 16)] = vals

    def field(f, ob):
        load_field_idx(f, ob)
        gat(0, 0, ob).start()
        for sub in range(_NSUB):
            gb = sub & 1
            gat(sub, gb, ob).wait()
            if sub + 1 < _NSUB:
                gat(sub + 1, 1 - gb, ob).start()
            extract(sub, gb, ob)
        @pl.when(f >= 2)
        def _():
            pltpu.make_async_copy(
                osb.at[ob], out_hbm.at[f - 2, :, pl.ds(b0, _BW)],
                osem.at[ob]).wait()
        outw(f, ob).start()

    def body(i, carry):
        f = i * 2
        field(f, 0)
        field(f + 1, 1)
        return carry

    lax.fori_loop(0, FIELDS // 2, body, jnp.int32(0))
    outw(FIELDS - 2, 0).wait()
    outw(FIELDS - 1, 1).wait()


def kernel(input, table):
    out = _gather_kernel(input.T, jnp.reshape(table, (ROWS4, 128)))
    return jnp.transpose(out, (2, 0, 1))


# speedup vs baseline: 1.4328x; 1.0504x over previous
import functools

import jax
import jax.numpy as jnp
from jax import lax
from jax.experimental import pallas as pl
from jax.experimental.pallas import tpu as pltpu
from jax.experimental.pallas import tpu_sc as plsc

BATCH = 16384
FIELDS = 26
OUT_DIM = 32
ROWS4 = 250000

_NC = 2
_NS = 16
_NW = _NC * _NS
_BW = BATCH // _NW
_SUB = 128
_NSUB = _BW // _SUB

_mesh = plsc.VectorSubcoreMesh(core_axis_name="c", subcore_axis_name="s")


@functools.partial(
    pl.kernel,
    mesh=_mesh,
    out_type=jax.ShapeDtypeStruct((FIELDS, OUT_DIM, BATCH), jnp.float32),
    scratch_types=[
        pltpu.VMEM((2, _BW), jnp.int32),
        pltpu.VMEM((2, _NSUB, _SUB), jnp.int32),
        pltpu.VMEM((_NSUB, _SUB, 128), jnp.float32),
        pltpu.VMEM((2, OUT_DIM, _BW), jnp.float32),
        pltpu.SemaphoreType.DMA((_NSUB,)),
        pltpu.SemaphoreType.DMA((2,)),
    ],
    compiler_params=pltpu.CompilerParams(
        use_tc_tiling_on_sc=True, needs_layout_passes=False),
)
def _gather_kernel(idx_hbm, table_hbm, out_hbm, idx_v, row_v, rows_v, osb,
                   gsem, osem):
    wid = lax.axis_index("s") * _NC + lax.axis_index("c")
    b0 = wid * _BW

    iota = lax.iota(jnp.int32, 16)

    def load_field_idx(f, ob):
        pltpu.sync_copy(idx_hbm.at[f, pl.ds(b0, _BW)], idx_v.at[ob])
        for sub in range(_NSUB):
            for g in range(_SUB // 16):
                v = idx_v[ob, pl.ds(sub * _SUB + g * 16, 16)]
                row_v[ob, sub, pl.ds(g * 16, 16)] = (
                    lax.shift_right_logical(v, 2))

    def gat(sub, ob):
        return pltpu.make_async_copy(
            table_hbm.at[row_v.at[ob, sub]], rows_v.at[sub], gsem.at[sub])

    def outw(f, ob):
        return pltpu.make_async_copy(
            osb.at[ob], out_hbm.at[f, :, pl.ds(b0, _BW)], osem.at[ob])

    def extract(sub, ob):
        base_b = sub * _SUB
        for g in range(_SUB // 16):
            idx16 = idx_v[ob, pl.ds(base_b + g * 16, 16)]
            lane0 = lax.shift_left(lax.bitwise_and(idx16, jnp.int32(3)),
                                   jnp.int32(5))
            rows16 = iota + jnp.int32(g * 16)
            col = lane0
            for d in range(OUT_DIM):
                vals = plsc.load_gather(rows_v.at[sub], [rows16, col])
                osb[ob, d, pl.ds(base_b + g * 16, 16)] = vals
                if d + 1 < OUT_DIM:
                    col = col + jnp.int32(1)

    load_field_idx(0, 0)
    for sub in range(_NSUB):
        gat(sub, 0).start()

    def body(f, carry):
        ob = lax.rem(f, 2)

        @pl.when(ob == 0)
        def _():
            field_step(f, 0)

        @pl.when(ob == 1)
        def _():
            field_step(f, 1)

        return carry

    def field_step(f, ob):
        nxt = 1 - ob
        @pl.when(f + 1 < FIELDS)
        def _():
            load_field_idx(f + 1, nxt)
        for sub in range(_NSUB):
            gat(sub, ob).wait()
            extract(sub, ob)

            @pl.when(f + 1 < FIELDS)
            def _():
                gat(sub, nxt).start()
        @pl.when(f >= 2)
        def _():
            pltpu.make_async_copy(
                osb.at[ob], out_hbm.at[f - 2, :, pl.ds(b0, _BW)],
                osem.at[ob]).wait()
        outw(f, ob).start()

    lax.fori_loop(0, FIELDS, body, jnp.int32(0))
    outw(FIELDS - 2, 0).wait()
    outw(FIELDS - 1, 1).wait()


def kernel(input, table):
    out = _gather_kernel(input.T, jnp.reshape(table, (ROWS4, 128)))
    return jnp.transpose(out, (2, 0, 1))


# speedup vs baseline: 1.5565x; 1.0864x over previous
import functools

import jax
import jax.numpy as jnp
from jax import lax
from jax.experimental import pallas as pl
from jax.experimental.pallas import tpu as pltpu
from jax.experimental.pallas import tpu_sc as plsc

BATCH = 16384
FIELDS = 26
OUT_DIM = 32

_NC = 2
_NS = 16
_NW = _NC * _NS
_BW = BATCH // _NW

_mesh = plsc.VectorSubcoreMesh(core_axis_name="c", subcore_axis_name="s")


@functools.partial(
    pl.kernel,
    mesh=_mesh,
    out_type=jax.ShapeDtypeStruct((BATCH, FIELDS, OUT_DIM), jnp.float32),
    scratch_types=[
        pltpu.VMEM((FIELDS, _BW), jnp.int32),
        pltpu.VMEM((2, _BW, OUT_DIM), jnp.float32),
        pltpu.SemaphoreType.DMA((2,)),
        pltpu.SemaphoreType.DMA((2,)),
    ],
    compiler_params=pltpu.CompilerParams(use_tc_tiling_on_sc=False),
)
def _gather_kernel(idx_hbm, table_hbm, out_hbm, idx_v, rows_v, gsem, osem):
    wid = lax.axis_index("s") * _NC + lax.axis_index("c")
    b0 = wid * _BW

    pltpu.sync_copy(idx_hbm.at[:, pl.ds(b0, _BW)], idx_v)

    def gat(f, b):
        return pltpu.make_async_copy(
            table_hbm.at[idx_v.at[f]], rows_v.at[b], gsem.at[b])

    def outc(f, b):
        return pltpu.make_async_copy(
            rows_v.at[b], out_hbm.at[pl.ds(b0, _BW), f, :], osem.at[b])

    gat(0, 0).start()
    for f in range(FIELDS):
        b = f & 1
        gat(f, b).wait()
        if f + 1 < FIELDS:
            if f >= 1:
                outc(f - 1, 1 - b).wait()
            gat(f + 1, 1 - b).start()
        outc(f, b).start()
    outc(FIELDS - 2, FIELDS & 1).wait()
    outc(FIELDS - 1, (FIELDS - 1) & 1).wait()


def kernel(input, table):
    return _gather_kernel(input.T, table)
